# baseline (speedup 1.0000x reference)
import math

import jax
import jax.numpy as jnp
from jax import lax
from jax.experimental import pallas as pl
from jax.experimental.pallas import tpu as pltpu
from jax.experimental.pallas import tpu_sc as plsc

B, L, V, D, O = 4096, 200, 1000000, 64, 32
NC, NS = 2, 16
NW = NC * NS
BPW = B // NW
C0 = 128
C1 = L - C0
ROW_UNROLL = 8


RB = 168
RPW = 31248
NBLK = RPW // RB
VREM = V - NW * RPW


def _sc_depad(table):
  mesh = plsc.VectorSubcoreMesh(core_axis_name="c", subcore_axis_name="s")

  def body(tab_hbm, out_hbm, a2, a1, b2, b1, sem_a, sem_b, sem_oa, sem_ob):
    wid = lax.axis_index("s") * NC + lax.axis_index("c")
    base = wid * RPW

    def issue(blk, buf, sem):
      pltpu.async_copy(tab_hbm.at[pl.ds(base + blk * RB, RB)], buf, sem)

    def wait_in(buf, sem):
      pltpu.make_async_copy(tab_hbm.at[pl.ds(0, RB)], buf, sem).wait()

    def repack(src2, dst1):
      def step(r, carry):
        for j in range(4):
          dst1[pl.ds(r * D + 16 * j, 16)] = src2[r, pl.ds(16 * j, 16)]
        return carry

      lax.fori_loop(0, RB, step, 0)

    def flush(blk, buf, sem):
      pltpu.async_copy(
          buf, out_hbm.at[pl.ds((base + blk * RB) * D, RB * D)], sem)

    def wait_out(buf, sem):
      pltpu.make_async_copy(buf, out_hbm.at[pl.ds(0, RB * D)], sem).wait()

    issue(0, a2, sem_a)

    def pair(p, carry):
      issue(2 * p + 1, b2, sem_b)
      wait_in(a2, sem_a)

      @pl.when(p > 0)
      def _():
        wait_out(a1, sem_oa)

      repack(a2, a1)
      flush(2 * p, a1, sem_oa)

      @pl.when(2 * p + 2 < NBLK)
      def _():
        issue(2 * p + 2, a2, sem_a)

      wait_in(b2, sem_b)

      @pl.when(p > 0)
      def _():
        wait_out(b1, sem_ob)

      repack(b2, b1)
      flush(2 * p + 1, b1, sem_ob)
      return carry

    lax.fori_loop(0, NBLK // 2, pair, 0)
    wait_out(a1, sem_oa)
    wait_out(b1, sem_ob)

    @pl.when(wid == 0)
    def _():
      rbase = NW * RPW
      pltpu.sync_copy(tab_hbm.at[pl.ds(rbase, VREM)], a2.at[pl.ds(0, VREM)])

      def step(r, carry):
        for j in range(4):
          a1[pl.ds(r * D + 16 * j, 16)] = a2[r, pl.ds(16 * j, 16)]
        return carry

      lax.fori_loop(0, VREM, step, 0)
      pltpu.sync_copy(a1.at[pl.ds(0, VREM * D)],
                      out_hbm.at[pl.ds(rbase * D, VREM * D)])

  return pl.kernel(
      body,
      out_type=jax.ShapeDtypeStruct((V * D,), jnp.float32),
      mesh=mesh,
      scratch_types=[
          pltpu.VMEM((RB, D), jnp.float32),
          pltpu.VMEM((RB * D,), jnp.float32),
          pltpu.VMEM((RB, D), jnp.float32),
          pltpu.VMEM((RB * D,), jnp.float32),
          pltpu.SemaphoreType.DMA,
          pltpu.SemaphoreType.DMA,
          pltpu.SemaphoreType.DMA,
          pltpu.SemaphoreType.DMA,
      ],
      compiler_params=pltpu.CompilerParams(use_tc_tiling_on_sc=True,
                                           needs_layout_passes=False),
  )(table)


def _sc_gather_sum(tokens_flat, table):
  mesh = plsc.VectorSubcoreMesh(core_axis_name="c", subcore_axis_name="s")

  def body(tokens_hbm, table_hbm, out_hbm, idx_v, buf_a, buf_b, out_v,
           sem_a, sem_b):
    wid = lax.axis_index("s") * NC + lax.axis_index("c")
    base = wid * BPW
    pltpu.sync_copy(tokens_hbm.at[pl.ds(base * L, BPW * L)], idx_v)

    def issue(bag, buf, sem):
      off = bag * L
      pltpu.async_copy(table_hbm.at[idx_v.at[pl.ds(off, C0)]],
                       buf.at[pl.ds(0, C0)], sem)
      pltpu.async_copy(table_hbm.at[idx_v.at[pl.ds(off + C0, C1)]],
                       buf.at[pl.ds(C0, C1)], sem)

    def wait(buf, sem):
      pltpu.make_async_copy(table_hbm.at[pl.ds(0, L)], buf, sem).wait()

    def accumulate(bag, buf):
      zeros = jnp.zeros((16,), jnp.float32)

      def step(i, accs):
        r = i * ROW_UNROLL
        new = list(accs)
        for dr in range(ROW_UNROLL):
          for j in range(4):
            new[j] = new[j] + buf[r + dr, pl.ds(16 * j, 16)]
        return tuple(new)

      accs = lax.fori_loop(0, L // ROW_UNROLL, step,
                           (zeros, zeros, zeros, zeros))
      for j in range(4):
        out_v[pl.ds(bag * D + 16 * j, 16)] = accs[j]

    issue(0, buf_a, sem_a)

    def pair(p, carry):
      bag = p * 2
      issue(bag + 1, buf_b, sem_b)
      wait(buf_a, sem_a)
      accumulate(bag, buf_a)

      @pl.when(bag + 2 < BPW)
      def _():
        issue(bag + 2, buf_a, sem_a)

      wait(buf_b, sem_b)
      accumulate(bag + 1, buf_b)
      return carry

    lax.fori_loop(0, BPW // 2, pair, 0)
    pltpu.sync_copy(out_v, out_hbm.at[pl.ds(base * D, BPW * D)])

  return pl.kernel(
      body,
      out_type=jax.ShapeDtypeStruct((B * D,), jnp.float32),
      mesh=mesh,
      scratch_types=[
          pltpu.VMEM((BPW * L,), jnp.int32),
          pltpu.VMEM((L, D), jnp.float32),
          pltpu.VMEM((L, D), jnp.float32),
          pltpu.VMEM((BPW * D,), jnp.float32),
          pltpu.SemaphoreType.DMA,
          pltpu.SemaphoreType.DMA,
      ],
      compiler_params=pltpu.CompilerParams(use_tc_tiling_on_sc=False),
  )(tokens_flat, table)


def _tc_head(tokens, sums, W1, b1, W2, b2):
  def body(tok_ref, sums_ref, w1_ref, b1_ref, w2_ref, b2_ref, out_ref):
    t = tok_ref[...]
    cnt = jnp.sum((t != 0).astype(jnp.float32), axis=1, keepdims=True)
    pooled = sums_ref[...] / jnp.maximum(cnt, 1.0)
    h = jnp.dot(pooled, w1_ref[...],
                preferred_element_type=jnp.float32) + b1_ref[...]
    h = 0.5 * h * (1.0 + lax.erf(h * (1.0 / math.sqrt(2.0))))
    out_ref[...] = jnp.dot(h, w2_ref[...],
                           preferred_element_type=jnp.float32) + b2_ref[...]

  grid = 8
  bb = B // grid
  return pl.pallas_call(
      body,
      out_shape=jax.ShapeDtypeStruct((B, O), jnp.float32),
      grid=(grid,),
      in_specs=[
          pl.BlockSpec((bb, L), lambda i: (i, 0)),
          pl.BlockSpec((bb, D), lambda i: (i, 0)),
          pl.BlockSpec((D, D), lambda i: (0, 0)),
          pl.BlockSpec((1, D), lambda i: (0, 0)),
          pl.BlockSpec((D, O), lambda i: (0, 0)),
          pl.BlockSpec((1, O), lambda i: (0, 0)),
      ],
      out_specs=pl.BlockSpec((bb, O), lambda i: (i, 0)),
  )(tokens, sums, W1, b1, W2, b2)


def kernel(tokens, table, W1, b1, W2, b2):
  tokens = tokens.astype(jnp.int32)
  table_lin = _sc_depad(table).reshape(V, D)
  sums = _sc_gather_sum(tokens.reshape(-1), table_lin).reshape(B, D)
  return _tc_head(tokens, sums, W1, b1.reshape(1, D), W2, b2.reshape(1, O))
